# baseline (speedup 1.0000x reference)
import functools

import jax
import jax.numpy as jnp
from jax import lax
from jax.experimental import pallas as pl
from jax.experimental.pallas import tpu as pltpu
from jax.experimental.pallas import tpu_sc as plsc

_V = 8192
_D = 8192
_B = 8192
_NC = 2
_NS = 16
_NW = _NC * _NS
_BW = _B // _NW
_R = 1
_NCHUNK = _BW // _R
_NSLOT = 8


def _gather_body(table_hbm, idx_hbm, out_hbm, idx_v, bufs, gsems, psems):
    wid = lax.axis_index("s") * _NC + lax.axis_index("c")
    base = wid * _BW
    pltpu.sync_copy(idx_hbm.at[wid], idx_v)

    def gather(chunk, k):
        pltpu.async_copy(table_hbm.at[idx_v.at[chunk]], bufs[k], gsems[k])

    def put(chunk, k):
        pltpu.async_copy(bufs[k], out_hbm.at[pl.ds(base + chunk, _R)],
                         psems[k])

    def wait_gather(k):
        pltpu.make_async_copy(table_hbm.at[pl.ds(0, _R)], bufs[k],
                              gsems[k]).wait()

    def wait_put(k):
        pltpu.make_async_copy(bufs[k], out_hbm.at[pl.ds(base, _R)],
                              psems[k]).wait()

    for k in range(_NSLOT):
        gather(k, k)

    @pl.loop(0, _NCHUNK - 2 * _NSLOT + 1, step=_NSLOT)
    def _body(i):
        for k in range(_NSLOT):
            wait_gather(k)
            put(i + k, k)
        for k in range(_NSLOT):
            wait_put(k)
            gather(i + _NSLOT + k, k)

    for k in range(_NSLOT):
        wait_gather(k)
        put(_NCHUNK - _NSLOT + k, k)
    for k in range(_NSLOT):
        wait_put(k)


@jax.jit
def _gather(table, idx):
    run = functools.partial(
        pl.kernel,
        mesh=plsc.VectorSubcoreMesh(core_axis_name="c", subcore_axis_name="s"),
        out_type=jax.ShapeDtypeStruct((_B, _D), jnp.float32),
        scratch_types=[
            pltpu.VMEM((_NCHUNK, _R), jnp.int32),
            [pltpu.VMEM((_R, _D), jnp.float32) for _ in range(_NSLOT)],
            [pltpu.SemaphoreType.DMA for _ in range(_NSLOT)],
            [pltpu.SemaphoreType.DMA for _ in range(_NSLOT)],
        ],
    )(_gather_body)
    return run(table, idx)


def kernel(x, table):
    idx = x.reshape(_NW, _NCHUNK, _R)
    out = _gather(table, idx)
    return out.reshape(x.shape + (table.shape[1],))
